# baseline (speedup 1.0000x reference)
import functools

import jax
import jax.numpy as jnp
from jax import lax
from jax.experimental import pallas as pl
from jax.experimental.pallas import tpu as pltpu
from jax.experimental.pallas import tpu_sc as plsc

N = 10000
E = 320000
F = 128
FD = 144
H = 256

NPAD = 10112
RPT = NPAD // 16
EPAD = 327680
EPT = EPAD // 32
CH = 128
NCHUNK = EPT // CH
IQ = 4


def _make_segsum(with_deg, pipelined, nb=2):
    mesh = plsc.VectorSubcoreMesh(core_axis_name="c", subcore_axis_name="s")
    if with_deg:
        out_type = [jax.ShapeDtypeStruct((2, NPAD, F), jnp.float32),
                    jax.ShapeDtypeStruct((2 * NPAD,), jnp.float32)]
    else:
        out_type = jax.ShapeDtypeStruct((2, NPAD, F), jnp.float32)
    if not pipelined:
        nb = 1
    scratch = [
        [pltpu.VMEM((2, CH), jnp.int32) for _ in range(IQ)],
        [pltpu.VMEM((CH, F), jnp.float32) for _ in range(nb)],
        pltpu.VMEM((CH,), jnp.float32),
        pltpu.VMEM((RPT,), jnp.float32),
        pltpu.VMEM_SHARED((NPAD, F), jnp.float32),
        pltpu.VMEM_SHARED((NPAD,), jnp.float32),
        [pltpu.SemaphoreType.DMA for _ in range(nb)],
        [pltpu.SemaphoreType.DMA for _ in range(nb)],
        [pltpu.SemaphoreType.DMA for _ in range(IQ)],
        pltpu.SemaphoreType.DMA,
    ]

    @functools.partial(pl.kernel, mesh=mesh, out_type=out_type,
                       scratch_types=scratch)
    def seg(x_hbm, eidx_hbm, zrows_hbm, *rest):
        if with_deg:
            out_hbm, deg_hbm = rest[0], rest[1]
            rest = rest[2:]
        else:
            out_hbm = rest[0]
            rest = rest[1:]
        idxq, rows, onesv, dv, acc, dacc, gsem, ssem, isem, dsem = rest
        c = lax.axis_index("c")
        s = lax.axis_index("s")
        wid = c * 16 + s

        pltpu.sync_copy(zrows_hbm, acc.at[pl.ds(s * RPT, RPT)])
        if with_deg:
            for k in range(RPT // 16):
                dv[pl.ds(k * 16, 16)] = jnp.zeros((16,), jnp.float32)
            if RPT % 16:
                dv[pl.ds(RPT - 16, 16)] = jnp.zeros((16,), jnp.float32)
            pltpu.sync_copy(dv, dacc.at[pl.ds(s * RPT, RPT)])
            for k in range(CH // 16):
                onesv[pl.ds(k * 16, 16)] = jnp.ones((16,), jnp.float32)
        plsc.subcore_barrier()

        def start_idx(j, q):
            pltpu.async_copy(eidx_hbm.at[wid * NCHUNK + j], idxq[q], isem[q])

        def wait_idx(q):
            pltpu.make_async_copy(eidx_hbm.at[0], idxq[q], isem[q]).wait()

        def start_gather(q, b):
            pltpu.async_copy(x_hbm.at[idxq[q].at[0]], rows[b], gsem[b])

        def wait_gather(b):
            pltpu.make_async_copy(x_hbm.at[pl.ds(0, CH)], rows[b],
                                  gsem[b]).wait()

        def start_scatter(q, b):
            pltpu.async_copy(rows[b], acc.at[idxq[q].at[1]], ssem[b],
                             add=True)
            if with_deg:
                pltpu.async_copy(onesv, dacc.at[idxq[q].at[1]], dsem,
                                 add=True)

        def wait_scatter(b):
            pltpu.make_async_copy(rows[b], acc.at[pl.ds(0, CH)],
                                  ssem[b]).wait()

        if not pipelined:
            for k in range(2):
                start_idx(k, k)

            def chunk(j, carry):
                wait_idx(0)
                start_gather(0, 0)

                @pl.when(j + 2 < NCHUNK)
                def _():
                    start_idx(j + 2, 0)
                wait_gather(0)
                start_scatter(0, 0)
                wait_scatter(0)
                return carry

            def chunk2(t, carry):
                for r in range(2):
                    j = 2 * t + r
                    q = r % IQ
                    wait_idx(q)
                    start_gather(q, 0)
                    wait_gather(0)
                    start_scatter(q, 0)

                    @pl.when(j + 2 < NCHUNK)
                    def _():
                        start_idx(j + 2, q)
                    wait_scatter(0)
                return carry

            lax.fori_loop(0, NCHUNK // 2, chunk2, 0)
        elif nb == 3:
            def step3(j, qj, b, skip_swait, do_gather, do_idx):
                bn = (b + 1) % 3
                wait_gather(b)
                start_scatter(qj, b)
                if not skip_swait:
                    wait_scatter(bn)
                if do_gather:
                    wait_idx((qj + 1) % IQ)
                    start_gather((qj + 1) % IQ, bn)
                if do_idx is not None:
                    start_idx(do_idx, (qj + 2) % IQ)

            start_idx(0, 0)
            start_idx(1, 1)
            wait_idx(0)
            start_gather(0, 0)
            for j in range(12):
                step3(j, j % IQ, j % 3, skip_swait=(j < 2),
                      do_gather=True, do_idx=j + 2)

            def outer3(t, carry):
                jbase = 12 * t + 12
                for r in range(12):
                    j = jbase + r
                    step3(j, r % IQ, r % 3, skip_swait=False,
                          do_gather=True, do_idx=j + 2)
                return carry

            lax.fori_loop(0, (NCHUNK - 20) // 12, outer3, 0)

            for j in range(NCHUNK - 8, NCHUNK):
                step3(j, j % IQ, j % 3, skip_swait=False,
                      do_gather=(j + 1 < NCHUNK),
                      do_idx=(j + 2) if j + 2 < NCHUNK else None)
            wait_scatter((NCHUNK - 2) % 3)
            wait_scatter((NCHUNK - 1) % 3)
        else:
            def body(j, qj, b, first, last):
                bo = 1 - b
                wait_gather(b)
                start_scatter(qj % IQ, b)
                if not first:
                    wait_scatter(bo)
                if not last:
                    wait_idx((qj + 1) % IQ)
                    start_gather((qj + 1) % IQ, bo)

            for k in range(3):
                start_idx(k, k)
            wait_idx(0)
            start_gather(0, 0)
            for j in (0, 1):
                body(j, j, j % 2, first=(j == 0), last=False)
                start_idx(j + 3, (j + 3) % IQ)

            def outer(t, carry):
                jbase = 4 * t + 2
                for r in range(4):
                    j = jbase + r
                    q = (2 + r) % IQ
                    b = (2 + r) % 2
                    bo = 1 - b
                    wait_gather(b)
                    start_scatter(q, b)
                    wait_scatter(bo)
                    wait_idx((q + 1) % IQ)
                    start_gather((q + 1) % IQ, bo)
                    qn = (q + 3) % IQ

                    @pl.when(j + 3 < NCHUNK)
                    def _():
                        start_idx(j + 3, qn)
                return carry

            lax.fori_loop(0, (NCHUNK - 4) // 4, outer, 0)

            j = NCHUNK - 2
            body(j, j, j % 2, first=False, last=False)
            j = NCHUNK - 1
            body(j, j, j % 2, first=False, last=True)
            wait_scatter(j % 2)

        if with_deg:
            def drain(j, carry):
                pltpu.make_async_copy(onesv, dacc.at[pl.ds(0, CH)],
                                      dsem).wait()
                return carry
            lax.fori_loop(0, NCHUNK, drain, 0)
        plsc.subcore_barrier()
        pltpu.sync_copy(acc.at[pl.ds(s * RPT, RPT)],
                        out_hbm.at[c, pl.ds(s * RPT, RPT)])
        if with_deg:
            pltpu.sync_copy(dacc.at[pl.ds(s * RPT, RPT)], dv)
            pltpu.sync_copy(dv, deg_hbm.at[pl.ds(c * NPAD + s * RPT, RPT)])

    return seg


_seg1 = _make_segsum(True, pipelined=True)
_seg2 = _make_segsum(False, pipelined=True, nb=3)

R = 1000
GRID = N // R


def _tc1_body(x_r, p1_r, degt_r, ws1_r, wn1_r, b1_r, wn2_r, ws2_r,
              b2_r, hs_r, y_r, rdegb_r):
    deg = degt_r[:, 0] + degt_r[:, 1]
    rdeg = 1.0 / jnp.maximum(deg, 1.0)
    mean1 = (p1_r[0] + p1_r[1]) * rdeg[:, None]
    h = x_r[...] @ ws1_r[...] + mean1 @ wn1_r[...] + b1_r[...]
    h = jnp.maximum(h, 0.0)
    hs_r[...] = h @ ws2_r[...] + b2_r[...]
    y_r[...] = h @ wn2_r[...]
    rdegb_r[...] = jnp.broadcast_to(rdeg[:, None], (R, F))


def _tc1(x, p1, degt, ws1, wn1, b1, wn2, ws2, b2):
    return pl.pallas_call(
        _tc1_body,
        grid=(GRID,),
        in_specs=[
            pl.BlockSpec((R, F), lambda i: (i, 0)),
            pl.BlockSpec((2, R, F), lambda i: (0, i, 0)),
            pl.BlockSpec((R, 2), lambda i: (i, 0)),
            pl.BlockSpec((F, H), lambda i: (0, 0)),
            pl.BlockSpec((F, H), lambda i: (0, 0)),
            pl.BlockSpec((1, H), lambda i: (0, 0)),
            pl.BlockSpec((H, F), lambda i: (0, 0)),
            pl.BlockSpec((H, F), lambda i: (0, 0)),
            pl.BlockSpec((1, F), lambda i: (0, 0)),
        ],
        out_specs=[
            pl.BlockSpec((R, F), lambda i: (i, 0)),
            pl.BlockSpec((R, F), lambda i: (i, 0)),
            pl.BlockSpec((R, F), lambda i: (i, 0)),
        ],
        out_shape=[
            jax.ShapeDtypeStruct((N, F), jnp.float32),
            jax.ShapeDtypeStruct((N, F), jnp.float32),
            jax.ShapeDtypeStruct((N, F), jnp.float32),
        ],
    )(x, p1, degt, ws1, wn1, b1, wn2, ws2, b2)


def _tc2_body(hs_r, p2_r, rdegb_r, out_r):
    out_r[...] = hs_r[...] + (p2_r[0] + p2_r[1]) * rdegb_r[...]


def _tc2(hs, p2, rdegb):
    return pl.pallas_call(
        _tc2_body,
        grid=(GRID,),
        in_specs=[
            pl.BlockSpec((R, F), lambda i: (i, 0)),
            pl.BlockSpec((2, R, F), lambda i: (0, i, 0)),
            pl.BlockSpec((R, F), lambda i: (i, 0)),
        ],
        out_specs=pl.BlockSpec((R, F), lambda i: (i, 0)),
        out_shape=jax.ShapeDtypeStruct((N, F), jnp.float32),
    )(hs, p2, rdegb)


def kernel(features, edge_index, W_self1, W_neigh1, b1, W_self2, W_neigh2, b2):
    src = edge_index[0].astype(jnp.int32)
    dst = edge_index[1].astype(jnp.int32)
    pad = EPAD - E
    fill = jnp.arange(pad, dtype=jnp.int32)
    src_p = jnp.concatenate([src, fill % N])
    dst_p = jnp.concatenate([dst, N + fill % (NPAD - N)])
    eidx = jnp.stack([src_p.reshape(32 * NCHUNK, CH),
                      dst_p.reshape(32 * NCHUNK, CH)], axis=1)
    zrows = jnp.zeros((RPT, F), jnp.float32)

    p1, pdeg = _seg1(features, eidx, zrows)
    degt = jnp.transpose(pdeg.reshape(2, NPAD)[:, :N])
    hs, y, rdegb = _tc1(features, p1, degt, W_self1,
                        W_neigh1, b1.reshape(1, H), W_neigh2, W_self2,
                        b2.reshape(1, F))

    p2 = _seg2(y, eidx, zrows)
    out = _tc2(hs, p2, rdegb)
    return out


# speedup vs baseline: 1.0009x; 1.0009x over previous
import functools

import jax
import jax.numpy as jnp
from jax import lax
from jax.experimental import pallas as pl
from jax.experimental.pallas import tpu as pltpu
from jax.experimental.pallas import tpu_sc as plsc

N = 10000
E = 320000
F = 128
FD = 144
H = 256

NPAD = 10112
RPT = NPAD // 16
EPAD = 327680
EPT = EPAD // 32
CH = 128
NCHUNK = EPT // CH
IQ = 4


def _make_segsum(with_deg, pipelined, nb=2):
    mesh = plsc.VectorSubcoreMesh(core_axis_name="c", subcore_axis_name="s")
    if with_deg:
        out_type = [jax.ShapeDtypeStruct((2, NPAD, F), jnp.float32),
                    jax.ShapeDtypeStruct((2 * NPAD,), jnp.float32)]
    else:
        out_type = jax.ShapeDtypeStruct((2, NPAD, F), jnp.float32)
    if not pipelined:
        nb = 1
    scratch = [
        [pltpu.VMEM((2, CH), jnp.int32) for _ in range(IQ)],
        [pltpu.VMEM((CH, F), jnp.float32) for _ in range(nb)],
        pltpu.VMEM((CH,), jnp.float32),
        pltpu.VMEM((RPT,), jnp.float32),
        pltpu.VMEM_SHARED((NPAD, F), jnp.float32),
        pltpu.VMEM_SHARED((NPAD,), jnp.float32),
        [pltpu.SemaphoreType.DMA for _ in range(nb)],
        [pltpu.SemaphoreType.DMA for _ in range(nb)],
        [pltpu.SemaphoreType.DMA for _ in range(IQ)],
        pltpu.SemaphoreType.DMA,
    ]

    @functools.partial(pl.kernel, mesh=mesh, out_type=out_type,
                       scratch_types=scratch)
    def seg(x_hbm, eidx_hbm, zrows_hbm, *rest):
        if with_deg:
            out_hbm, deg_hbm = rest[0], rest[1]
            rest = rest[2:]
        else:
            out_hbm = rest[0]
            rest = rest[1:]
        idxq, rows, onesv, dv, acc, dacc, gsem, ssem, isem, dsem = rest
        c = lax.axis_index("c")
        s = lax.axis_index("s")
        wid = c * 16 + s

        pltpu.sync_copy(zrows_hbm, acc.at[pl.ds(s * RPT, RPT)])
        if with_deg:
            for k in range(RPT // 16):
                dv[pl.ds(k * 16, 16)] = jnp.zeros((16,), jnp.float32)
            if RPT % 16:
                dv[pl.ds(RPT - 16, 16)] = jnp.zeros((16,), jnp.float32)
            pltpu.sync_copy(dv, dacc.at[pl.ds(s * RPT, RPT)])
            for k in range(CH // 16):
                onesv[pl.ds(k * 16, 16)] = jnp.ones((16,), jnp.float32)
        plsc.subcore_barrier()

        def start_idx(j, q):
            pltpu.async_copy(eidx_hbm.at[wid * NCHUNK + j], idxq[q], isem[q])

        def wait_idx(q):
            pltpu.make_async_copy(eidx_hbm.at[0], idxq[q], isem[q]).wait()

        def start_gather(q, b):
            pltpu.async_copy(x_hbm.at[idxq[q].at[0]], rows[b], gsem[b])

        def wait_gather(b):
            pltpu.make_async_copy(x_hbm.at[pl.ds(0, CH)], rows[b],
                                  gsem[b]).wait()

        def start_scatter(q, b):
            pltpu.async_copy(rows[b], acc.at[idxq[q].at[1]], ssem[b],
                             add=True)
            if with_deg:
                pltpu.async_copy(onesv, dacc.at[idxq[q].at[1]], dsem,
                                 add=True)

        def wait_scatter(b):
            pltpu.make_async_copy(rows[b], acc.at[pl.ds(0, CH)],
                                  ssem[b]).wait()

        if not pipelined:
            for k in range(2):
                start_idx(k, k)

            def chunk(j, carry):
                wait_idx(0)
                start_gather(0, 0)

                @pl.when(j + 2 < NCHUNK)
                def _():
                    start_idx(j + 2, 0)
                wait_gather(0)
                start_scatter(0, 0)
                wait_scatter(0)
                return carry

            def chunk2(t, carry):
                for r in range(2):
                    j = 2 * t + r
                    q = r % IQ
                    wait_idx(q)
                    start_gather(q, 0)
                    wait_gather(0)
                    start_scatter(q, 0)

                    @pl.when(j + 2 < NCHUNK)
                    def _():
                        start_idx(j + 2, q)
                    wait_scatter(0)
                return carry

            lax.fori_loop(0, NCHUNK // 2, chunk2, 0)
        elif nb == 3:
            def step3(j, qj, b, skip_swait, do_gather, do_idx):
                bn = (b + 1) % 3
                wait_gather(b)
                start_scatter(qj, b)
                if not skip_swait:
                    wait_scatter(bn)
                if do_gather:
                    wait_idx((qj + 1) % IQ)
                    start_gather((qj + 1) % IQ, bn)
                if do_idx is not None:
                    start_idx(do_idx, (qj + 2) % IQ)

            start_idx(0, 0)
            start_idx(1, 1)
            wait_idx(0)
            start_gather(0, 0)
            for j in range(12):
                step3(j, j % IQ, j % 3, skip_swait=(j < 2),
                      do_gather=True, do_idx=j + 2)

            def outer3(t, carry):
                jbase = 12 * t + 12
                for r in range(12):
                    j = jbase + r
                    step3(j, r % IQ, r % 3, skip_swait=False,
                          do_gather=True, do_idx=j + 2)
                return carry

            lax.fori_loop(0, (NCHUNK - 20) // 12, outer3, 0)

            for j in range(NCHUNK - 8, NCHUNK):
                step3(j, j % IQ, j % 3, skip_swait=False,
                      do_gather=(j + 1 < NCHUNK),
                      do_idx=(j + 2) if j + 2 < NCHUNK else None)
            wait_scatter((NCHUNK - 2) % 3)
            wait_scatter((NCHUNK - 1) % 3)
        else:
            def body(j, qj, b, first, last):
                bo = 1 - b
                wait_gather(b)
                start_scatter(qj % IQ, b)
                if not first:
                    wait_scatter(bo)
                if not last:
                    wait_idx((qj + 1) % IQ)
                    start_gather((qj + 1) % IQ, bo)

            for k in range(3):
                start_idx(k, k)
            wait_idx(0)
            start_gather(0, 0)
            for j in (0, 1):
                body(j, j, j % 2, first=(j == 0), last=False)
                start_idx(j + 3, (j + 3) % IQ)

            def outer(t, carry):
                jbase = 4 * t + 2
                for r in range(4):
                    j = jbase + r
                    q = (2 + r) % IQ
                    b = (2 + r) % 2
                    bo = 1 - b
                    wait_gather(b)
                    start_scatter(q, b)
                    wait_scatter(bo)
                    wait_idx((q + 1) % IQ)
                    start_gather((q + 1) % IQ, bo)
                    qn = (q + 3) % IQ

                    @pl.when(j + 3 < NCHUNK)
                    def _():
                        start_idx(j + 3, qn)
                return carry

            lax.fori_loop(0, (NCHUNK - 4) // 4, outer, 0)

            j = NCHUNK - 2
            body(j, j, j % 2, first=False, last=False)
            j = NCHUNK - 1
            body(j, j, j % 2, first=False, last=True)
            wait_scatter(j % 2)

        if with_deg:
            def drain(j, carry):
                pltpu.make_async_copy(onesv, dacc.at[pl.ds(0, CH)],
                                      dsem).wait()
                return carry
            lax.fori_loop(0, NCHUNK, drain, 0)
        plsc.subcore_barrier()
        pltpu.sync_copy(acc.at[pl.ds(s * RPT, RPT)],
                        out_hbm.at[c, pl.ds(s * RPT, RPT)])
        if with_deg:
            pltpu.sync_copy(dacc.at[pl.ds(s * RPT, RPT)], dv)
            pltpu.sync_copy(dv, deg_hbm.at[pl.ds(c * NPAD + s * RPT, RPT)])

    return seg


_seg1 = _make_segsum(True, pipelined=True)
_seg2 = _make_segsum(False, pipelined=True, nb=3)

R = 1000
GRID = N // R


def _tc1_body(x_r, p1_r, degt_r, ws1_r, wn1_r, b1_r, wn2_r, ws2_r,
              b2_r, hs_r, y_r):
    deg = degt_r[:, 0] + degt_r[:, 1]
    rdeg = 1.0 / jnp.maximum(deg, 1.0)
    mean1 = (p1_r[0] + p1_r[1]) * rdeg[:, None]
    h = x_r[...] @ ws1_r[...] + mean1 @ wn1_r[...] + b1_r[...]
    h = jnp.maximum(h, 0.0)
    hs_r[...] = h @ ws2_r[...] + b2_r[...]
    y_r[...] = h @ wn2_r[...]


def _tc1(x, p1, degt, ws1, wn1, b1, wn2, ws2, b2):
    return pl.pallas_call(
        _tc1_body,
        grid=(GRID,),
        in_specs=[
            pl.BlockSpec((R, F), lambda i: (i, 0)),
            pl.BlockSpec((2, R, F), lambda i: (0, i, 0)),
            pl.BlockSpec((R, 2), lambda i: (i, 0)),
            pl.BlockSpec((F, H), lambda i: (0, 0)),
            pl.BlockSpec((F, H), lambda i: (0, 0)),
            pl.BlockSpec((1, H), lambda i: (0, 0)),
            pl.BlockSpec((H, F), lambda i: (0, 0)),
            pl.BlockSpec((H, F), lambda i: (0, 0)),
            pl.BlockSpec((1, F), lambda i: (0, 0)),
        ],
        out_specs=[
            pl.BlockSpec((R, F), lambda i: (i, 0)),
            pl.BlockSpec((R, F), lambda i: (i, 0)),
        ],
        out_shape=[
            jax.ShapeDtypeStruct((N, F), jnp.float32),
            jax.ShapeDtypeStruct((N, F), jnp.float32),
        ],
    )(x, p1, degt, ws1, wn1, b1, wn2, ws2, b2)


def _tc2_body(hs_r, p2_r, degt_r, out_r):
    deg = degt_r[:, 0] + degt_r[:, 1]
    rdeg = 1.0 / jnp.maximum(deg, 1.0)
    out_r[...] = hs_r[...] + (p2_r[0] + p2_r[1]) * rdeg[:, None]


def _tc2(hs, p2, degt):
    return pl.pallas_call(
        _tc2_body,
        grid=(GRID,),
        in_specs=[
            pl.BlockSpec((R, F), lambda i: (i, 0)),
            pl.BlockSpec((2, R, F), lambda i: (0, i, 0)),
            pl.BlockSpec((R, 2), lambda i: (i, 0)),
        ],
        out_specs=pl.BlockSpec((R, F), lambda i: (i, 0)),
        out_shape=jax.ShapeDtypeStruct((N, F), jnp.float32),
    )(hs, p2, degt)


def kernel(features, edge_index, W_self1, W_neigh1, b1, W_self2, W_neigh2, b2):
    src = edge_index[0].astype(jnp.int32)
    dst = edge_index[1].astype(jnp.int32)
    pad = EPAD - E
    fill = jnp.arange(pad, dtype=jnp.int32)
    src_p = jnp.concatenate([src, fill % N])
    dst_p = jnp.concatenate([dst, N + fill % (NPAD - N)])
    eidx = jnp.stack([src_p.reshape(32 * NCHUNK, CH),
                      dst_p.reshape(32 * NCHUNK, CH)], axis=1)
    zrows = jnp.zeros((RPT, F), jnp.float32)

    p1, pdeg = _seg1(features, eidx, zrows)
    degt = jnp.transpose(pdeg.reshape(2, NPAD)[:, :N])
    hs, y = _tc1(features, p1, degt, W_self1,
                 W_neigh1, b1.reshape(1, H), W_neigh2, W_self2,
                 b2.reshape(1, F))

    p2 = _seg2(y, eidx, zrows)
    out = _tc2(hs, p2, degt)
    return out


# speedup vs baseline: 1.0024x; 1.0015x over previous
import functools

import jax
import jax.numpy as jnp
from jax import lax
from jax.experimental import pallas as pl
from jax.experimental.pallas import tpu as pltpu
from jax.experimental.pallas import tpu_sc as plsc

N = 10000
E = 320000
F = 128
FD = 144
H = 256

NPAD = 10112
RPT = NPAD // 16
EPAD = 327680
EPT = EPAD // 32
CH = 128
NCHUNK = EPT // CH
IQ = 4


def _make_segsum(with_deg, pipelined, nb=2):
    mesh = plsc.VectorSubcoreMesh(core_axis_name="c", subcore_axis_name="s")
    if with_deg:
        out_type = [jax.ShapeDtypeStruct((2, NPAD, F), jnp.float32),
                    jax.ShapeDtypeStruct((2 * NPAD,), jnp.float32)]
    else:
        out_type = jax.ShapeDtypeStruct((2, NPAD, F), jnp.float32)
    if not pipelined:
        nb = 1
    scratch = [
        [pltpu.VMEM((CH,), jnp.int32) for _ in range(IQ)],
        [pltpu.VMEM((CH,), jnp.int32) for _ in range(IQ)],
        [pltpu.VMEM((CH, F), jnp.float32) for _ in range(nb)],
        pltpu.VMEM((CH,), jnp.float32),
        pltpu.VMEM((RPT,), jnp.float32),
        pltpu.VMEM_SHARED((NPAD, F), jnp.float32),
        pltpu.VMEM_SHARED((NPAD,), jnp.float32),
        [pltpu.SemaphoreType.DMA for _ in range(nb)],
        [pltpu.SemaphoreType.DMA for _ in range(nb)],
        [pltpu.SemaphoreType.DMA for _ in range(IQ)],
        pltpu.SemaphoreType.DMA,
    ]

    @functools.partial(pl.kernel, mesh=mesh, out_type=out_type,
                       scratch_types=scratch)
    def seg(x_hbm, src_hbm, dst_hbm, zrows_hbm, *rest):
        if with_deg:
            out_hbm, deg_hbm = rest[0], rest[1]
            rest = rest[2:]
        else:
            out_hbm = rest[0]
            rest = rest[1:]
        srcq, dstq, rows, onesv, dv, acc, dacc, gsem, ssem, isem, dsem = rest
        c = lax.axis_index("c")
        s = lax.axis_index("s")
        wid = c * 16 + s

        pltpu.sync_copy(zrows_hbm, acc.at[pl.ds(s * RPT, RPT)])
        if with_deg:
            for k in range(RPT // 16):
                dv[pl.ds(k * 16, 16)] = jnp.zeros((16,), jnp.float32)
            if RPT % 16:
                dv[pl.ds(RPT - 16, 16)] = jnp.zeros((16,), jnp.float32)
            pltpu.sync_copy(dv, dacc.at[pl.ds(s * RPT, RPT)])
            for k in range(CH // 16):
                onesv[pl.ds(k * 16, 16)] = jnp.ones((16,), jnp.float32)
        plsc.subcore_barrier()

        def start_idx(j, q):
            pltpu.async_copy(src_hbm.at[wid * NCHUNK + j], srcq[q], isem[q])
            pltpu.async_copy(dst_hbm.at[wid * NCHUNK + j], dstq[q], isem[q])

        def wait_idx(q):
            pltpu.make_async_copy(src_hbm.at[0], srcq[q], isem[q]).wait()
            pltpu.make_async_copy(dst_hbm.at[0], dstq[q], isem[q]).wait()

        def start_gather(q, b):
            pltpu.async_copy(x_hbm.at[srcq[q]], rows[b], gsem[b])

        def wait_gather(b):
            pltpu.make_async_copy(x_hbm.at[pl.ds(0, CH)], rows[b],
                                  gsem[b]).wait()

        def start_scatter(q, b):
            pltpu.async_copy(rows[b], acc.at[dstq[q]], ssem[b], add=True)
            if with_deg:
                pltpu.async_copy(onesv, dacc.at[dstq[q]], dsem, add=True)

        def wait_scatter(b):
            pltpu.make_async_copy(rows[b], acc.at[pl.ds(0, CH)],
                                  ssem[b]).wait()

        if not pipelined:
            for k in range(2):
                start_idx(k, k)

            def chunk(j, carry):
                wait_idx(0)
                start_gather(0, 0)

                @pl.when(j + 2 < NCHUNK)
                def _():
                    start_idx(j + 2, 0)
                wait_gather(0)
                start_scatter(0, 0)
                wait_scatter(0)
                return carry

            def chunk2(t, carry):
                for r in range(2):
                    j = 2 * t + r
                    q = r % IQ
                    wait_idx(q)
                    start_gather(q, 0)
                    wait_gather(0)
                    start_scatter(q, 0)

                    @pl.when(j + 2 < NCHUNK)
                    def _():
                        start_idx(j + 2, q)
                    wait_scatter(0)
                return carry

            lax.fori_loop(0, NCHUNK // 2, chunk2, 0)
        elif nb == 3:
            def step3(j, qj, b, skip_swait, do_gather, do_idx):
                bn = (b + 1) % 3
                wait_gather(b)
                start_scatter(qj, b)
                if not skip_swait:
                    wait_scatter(bn)
                if do_gather:
                    wait_idx((qj + 1) % IQ)
                    start_gather((qj + 1) % IQ, bn)
                if do_idx is not None:
                    start_idx(do_idx, (qj + 2) % IQ)

            start_idx(0, 0)
            start_idx(1, 1)
            wait_idx(0)
            start_gather(0, 0)
            for j in range(12):
                step3(j, j % IQ, j % 3, skip_swait=(j < 2),
                      do_gather=True, do_idx=j + 2)

            def outer3(t, carry):
                jbase = 12 * t + 12
                for r in range(12):
                    j = jbase + r
                    step3(j, r % IQ, r % 3, skip_swait=False,
                          do_gather=True, do_idx=j + 2)
                return carry

            lax.fori_loop(0, (NCHUNK - 20) // 12, outer3, 0)

            for j in range(NCHUNK - 8, NCHUNK):
                step3(j, j % IQ, j % 3, skip_swait=False,
                      do_gather=(j + 1 < NCHUNK),
                      do_idx=(j + 2) if j + 2 < NCHUNK else None)
            wait_scatter((NCHUNK - 2) % 3)
            wait_scatter((NCHUNK - 1) % 3)
        else:
            def body(j, qj, b, first, last):
                bo = 1 - b
                wait_gather(b)
                start_scatter(qj % IQ, b)
                if not first:
                    wait_scatter(bo)
                if not last:
                    wait_idx((qj + 1) % IQ)
                    start_gather((qj + 1) % IQ, bo)

            for k in range(3):
                start_idx(k, k)
            wait_idx(0)
            start_gather(0, 0)
            for j in (0, 1):
                body(j, j, j % 2, first=(j == 0), last=False)
                start_idx(j + 3, (j + 3) % IQ)

            def outer(t, carry):
                jbase = 4 * t + 2
                for r in range(4):
                    j = jbase + r
                    q = (2 + r) % IQ
                    b = (2 + r) % 2
                    bo = 1 - b
                    wait_gather(b)
                    start_scatter(q, b)
                    wait_scatter(bo)
                    wait_idx((q + 1) % IQ)
                    start_gather((q + 1) % IQ, bo)
                    qn = (q + 3) % IQ

                    @pl.when(j + 3 < NCHUNK)
                    def _():
                        start_idx(j + 3, qn)
                return carry

            lax.fori_loop(0, (NCHUNK - 4) // 4, outer, 0)

            j = NCHUNK - 2
            body(j, j, j % 2, first=False, last=False)
            j = NCHUNK - 1
            body(j, j, j % 2, first=False, last=True)
            wait_scatter(j % 2)

        if with_deg:
            def drain(j, carry):
                pltpu.make_async_copy(onesv, dacc.at[pl.ds(0, CH)],
                                      dsem).wait()
                return carry
            lax.fori_loop(0, NCHUNK, drain, 0)
        plsc.subcore_barrier()
        pltpu.sync_copy(acc.at[pl.ds(s * RPT, RPT)],
                        out_hbm.at[c, pl.ds(s * RPT, RPT)])
        if with_deg:
            pltpu.sync_copy(dacc.at[pl.ds(s * RPT, RPT)], dv)
            pltpu.sync_copy(dv, deg_hbm.at[pl.ds(c * NPAD + s * RPT, RPT)])

    return seg


_seg1 = _make_segsum(True, pipelined=True)
_seg2 = _make_segsum(False, pipelined=True, nb=3)

R = 1000
GRID = N // R


def _tc1_body(x_r, p1_r, degt_r, ws1_r, wn1_r, b1_r, wn2_r, ws2_r,
              b2_r, hs_r, y_r):
    deg = degt_r[:, 0] + degt_r[:, 1]
    rdeg = 1.0 / jnp.maximum(deg, 1.0)
    mean1 = (p1_r[0] + p1_r[1]) * rdeg[:, None]
    h = x_r[...] @ ws1_r[...] + mean1 @ wn1_r[...] + b1_r[...]
    h = jnp.maximum(h, 0.0)
    hs_r[...] = h @ ws2_r[...] + b2_r[...]
    y_r[...] = h @ wn2_r[...]


def _tc1(x, p1, degt, ws1, wn1, b1, wn2, ws2, b2):
    return pl.pallas_call(
        _tc1_body,
        grid=(GRID,),
        in_specs=[
            pl.BlockSpec((R, F), lambda i: (i, 0)),
            pl.BlockSpec((2, R, F), lambda i: (0, i, 0)),
            pl.BlockSpec((R, 2), lambda i: (i, 0)),
            pl.BlockSpec((F, H), lambda i: (0, 0)),
            pl.BlockSpec((F, H), lambda i: (0, 0)),
            pl.BlockSpec((1, H), lambda i: (0, 0)),
            pl.BlockSpec((H, F), lambda i: (0, 0)),
            pl.BlockSpec((H, F), lambda i: (0, 0)),
            pl.BlockSpec((1, F), lambda i: (0, 0)),
        ],
        out_specs=[
            pl.BlockSpec((R, F), lambda i: (i, 0)),
            pl.BlockSpec((R, F), lambda i: (i, 0)),
        ],
        out_shape=[
            jax.ShapeDtypeStruct((N, F), jnp.float32),
            jax.ShapeDtypeStruct((N, F), jnp.float32),
        ],
    )(x, p1, degt, ws1, wn1, b1, wn2, ws2, b2)


def _tc2_body(hs_r, p2_r, degt_r, out_r):
    deg = degt_r[:, 0] + degt_r[:, 1]
    rdeg = 1.0 / jnp.maximum(deg, 1.0)
    out_r[...] = hs_r[...] + (p2_r[0] + p2_r[1]) * rdeg[:, None]


def _tc2(hs, p2, degt):
    return pl.pallas_call(
        _tc2_body,
        grid=(GRID,),
        in_specs=[
            pl.BlockSpec((R, F), lambda i: (i, 0)),
            pl.BlockSpec((2, R, F), lambda i: (0, i, 0)),
            pl.BlockSpec((R, 2), lambda i: (i, 0)),
        ],
        out_specs=pl.BlockSpec((R, F), lambda i: (i, 0)),
        out_shape=jax.ShapeDtypeStruct((N, F), jnp.float32),
    )(hs, p2, degt)


def kernel(features, edge_index, W_self1, W_neigh1, b1, W_self2, W_neigh2, b2):
    src = edge_index[0].astype(jnp.int32)
    dst = edge_index[1].astype(jnp.int32)
    pad = EPAD - E
    fill = jnp.arange(pad, dtype=jnp.int32)
    src_p = jnp.concatenate([src, fill % N])
    dst_p = jnp.concatenate([dst, N + fill % (NPAD - N)])
    src2 = src_p.reshape(32 * NCHUNK, CH)
    dst2 = dst_p.reshape(32 * NCHUNK, CH)
    zrows = jnp.zeros((RPT, F), jnp.float32)

    p1, pdeg = _seg1(features, src2, dst2, zrows)
    degt = jnp.transpose(pdeg.reshape(2, NPAD)[:, :N])
    hs, y = _tc1(features, p1, degt, W_self1,
                 W_neigh1, b1.reshape(1, H), W_neigh2, W_self2,
                 b2.reshape(1, F))

    p2 = _seg2(y, src2, dst2, zrows)
    out = _tc2(hs, p2, degt)
    return out


# speedup vs baseline: 1.0234x; 1.0210x over previous
import functools

import jax
import jax.numpy as jnp
from jax import lax
from jax.experimental import pallas as pl
from jax.experimental.pallas import tpu as pltpu
from jax.experimental.pallas import tpu_sc as plsc

N = 10000
E = 320000
F = 128
FD = 144
H = 256

NPAD = 10112
RPT = NPAD // 16
EPT = E // 32
CH = 128
NCHUNK = EPT // CH
TAILN = EPT - NCHUNK * CH
IQ = 4


def _make_segsum(with_deg):
    mesh = plsc.VectorSubcoreMesh(core_axis_name="c", subcore_axis_name="s")
    if with_deg:
        out_type = [jax.ShapeDtypeStruct((2, NPAD, F), jnp.float32),
                    jax.ShapeDtypeStruct((2 * NPAD,), jnp.float32)]
    else:
        out_type = jax.ShapeDtypeStruct((2, NPAD, F), jnp.float32)
    nb = 2
    scratch = [
        [pltpu.VMEM((CH,), jnp.int32) for _ in range(IQ)],
        [pltpu.VMEM((CH,), jnp.int32) for _ in range(IQ)],
        [pltpu.VMEM((CH, F), jnp.float32) for _ in range(nb)],
        pltpu.VMEM((TAILN,), jnp.int32),
        pltpu.VMEM((TAILN,), jnp.int32),
        pltpu.VMEM((TAILN, F), jnp.float32),
        pltpu.VMEM((CH,), jnp.float32),
        pltpu.VMEM((RPT,), jnp.float32),
        pltpu.VMEM_SHARED((NPAD, F), jnp.float32),
        pltpu.VMEM_SHARED((NPAD,), jnp.float32),
        [pltpu.SemaphoreType.DMA for _ in range(nb)],
        [pltpu.SemaphoreType.DMA for _ in range(nb)],
        [pltpu.SemaphoreType.DMA for _ in range(IQ)],
        pltpu.SemaphoreType.DMA,
    ]

    @functools.partial(pl.kernel, mesh=mesh, out_type=out_type,
                       scratch_types=scratch)
    def seg(x_hbm, src_hbm, dst_hbm, zrows_hbm, *rest):
        if with_deg:
            out_hbm, deg_hbm = rest[0], rest[1]
            rest = rest[2:]
        else:
            out_hbm = rest[0]
            rest = rest[1:]
        (srcq, dstq, rows, srct, dstt, rowst, onesv, dv, acc, dacc,
         gsem, ssem, isem, dsem) = rest
        c = lax.axis_index("c")
        s = lax.axis_index("s")
        wid = c * 16 + s
        ebase = wid * EPT

        pltpu.sync_copy(zrows_hbm, acc.at[pl.ds(s * RPT, RPT)])
        if with_deg:
            for k in range(RPT // 16):
                dv[pl.ds(k * 16, 16)] = jnp.zeros((16,), jnp.float32)
            if RPT % 16:
                dv[pl.ds(RPT - 16, 16)] = jnp.zeros((16,), jnp.float32)
            pltpu.sync_copy(dv, dacc.at[pl.ds(s * RPT, RPT)])
            for k in range(CH // 16):
                onesv[pl.ds(k * 16, 16)] = jnp.ones((16,), jnp.float32)
        plsc.subcore_barrier()

        def start_idx(j, q):
            b = pl.multiple_of(ebase + j * CH, 16)
            pltpu.async_copy(src_hbm.at[pl.ds(b, CH)], srcq[q], isem[q])
            pltpu.async_copy(dst_hbm.at[pl.ds(b, CH)], dstq[q], isem[q])

        def wait_idx(q):
            pltpu.make_async_copy(src_hbm.at[pl.ds(0, CH)], srcq[q],
                                  isem[q]).wait()
            pltpu.make_async_copy(dst_hbm.at[pl.ds(0, CH)], dstq[q],
                                  isem[q]).wait()

        def start_gather(q, b):
            pltpu.async_copy(x_hbm.at[srcq[q]], rows[b], gsem[b])

        def wait_gather(b):
            pltpu.make_async_copy(x_hbm.at[pl.ds(0, CH)], rows[b],
                                  gsem[b]).wait()

        def start_scatter(q, b):
            pltpu.async_copy(rows[b], acc.at[dstq[q]], ssem[b], add=True)
            if with_deg:
                pltpu.async_copy(onesv, dacc.at[dstq[q]], dsem, add=True)

        def wait_scatter(b):
            pltpu.make_async_copy(rows[b], acc.at[pl.ds(0, CH)],
                                  ssem[b]).wait()

        tb = pl.multiple_of(ebase + NCHUNK * CH, 16)
        pltpu.sync_copy(src_hbm.at[pl.ds(tb, TAILN)], srct)
        pltpu.sync_copy(dst_hbm.at[pl.ds(tb, TAILN)], dstt)
        pltpu.async_copy(x_hbm.at[srct], rowst, gsem[0]).wait()
        pltpu.async_copy(rowst, acc.at[dstt], ssem[0], add=True).wait()
        if with_deg:
            pltpu.async_copy(onesv.at[pl.ds(0, TAILN)], dacc.at[dstt],
                             dsem, add=True).wait()

        def body(j, qj, b, first, last):
            bo = 1 - b
            wait_gather(b)
            start_scatter(qj % IQ, b)
            if not first:
                wait_scatter(bo)
            if not last:
                wait_idx((qj + 1) % IQ)
                start_gather((qj + 1) % IQ, bo)

        for k in range(3):
            start_idx(k, k)
        wait_idx(0)
        start_gather(0, 0)
        for j in (0, 1):
            body(j, j, j % 2, first=(j == 0), last=False)
            start_idx(j + 3, (j + 3) % IQ)

        core_len = ((NCHUNK - 2 - 4) // 4) * 4

        def outer(t, carry):
            jbase = 4 * t + 2
            for r in range(4):
                j = jbase + r
                q = (2 + r) % IQ
                b = (2 + r) % 2
                bo = 1 - b
                wait_gather(b)
                start_scatter(q, b)
                wait_scatter(bo)
                wait_idx((q + 1) % IQ)
                start_gather((q + 1) % IQ, bo)
                start_idx(j + 3, (q + 3) % IQ)
            return carry

        lax.fori_loop(0, core_len // 4, outer, 0)

        for j in range(2 + core_len, NCHUNK):
            body(j, j, j % 2, first=False, last=(j == NCHUNK - 1))
            if j + 3 < NCHUNK:
                start_idx(j + 3, (j + 3) % IQ)
        wait_scatter((NCHUNK - 1) % 2)

        if with_deg:
            def drain(j, carry):
                pltpu.make_async_copy(onesv, dacc.at[pl.ds(0, CH)],
                                      dsem).wait()
                return carry
            lax.fori_loop(0, NCHUNK, drain, 0)
        plsc.subcore_barrier()
        pltpu.sync_copy(acc.at[pl.ds(s * RPT, RPT)],
                        out_hbm.at[c, pl.ds(s * RPT, RPT)])
        if with_deg:
            pltpu.sync_copy(dacc.at[pl.ds(s * RPT, RPT)], dv)
            pltpu.sync_copy(dv, deg_hbm.at[pl.ds(c * NPAD + s * RPT, RPT)])

    return seg


_seg1 = _make_segsum(True)
_seg2 = _make_segsum(False)

R = 1000
GRID = N // R


def _tc1_body(x_r, p1_r, degt_r, ws1_r, wn1_r, b1_r, wn2_r, ws2_r,
              b2_r, hs_r, y_r):
    deg = degt_r[:, 0] + degt_r[:, 1]
    rdeg = 1.0 / jnp.maximum(deg, 1.0)
    mean1 = (p1_r[0] + p1_r[1]) * rdeg[:, None]
    h = x_r[...] @ ws1_r[...] + mean1 @ wn1_r[...] + b1_r[...]
    h = jnp.maximum(h, 0.0)
    hs_r[...] = h @ ws2_r[...] + b2_r[...]
    y_r[...] = h @ wn2_r[...]


def _tc1(x, p1, degt, ws1, wn1, b1, wn2, ws2, b2):
    return pl.pallas_call(
        _tc1_body,
        grid=(GRID,),
        in_specs=[
            pl.BlockSpec((R, F), lambda i: (i, 0)),
            pl.BlockSpec((2, R, F), lambda i: (0, i, 0)),
            pl.BlockSpec((R, 2), lambda i: (i, 0)),
            pl.BlockSpec((F, H), lambda i: (0, 0)),
            pl.BlockSpec((F, H), lambda i: (0, 0)),
            pl.BlockSpec((1, H), lambda i: (0, 0)),
            pl.BlockSpec((H, F), lambda i: (0, 0)),
            pl.BlockSpec((H, F), lambda i: (0, 0)),
            pl.BlockSpec((1, F), lambda i: (0, 0)),
        ],
        out_specs=[
            pl.BlockSpec((R, F), lambda i: (i, 0)),
            pl.BlockSpec((R, F), lambda i: (i, 0)),
        ],
        out_shape=[
            jax.ShapeDtypeStruct((N, F), jnp.float32),
            jax.ShapeDtypeStruct((N, F), jnp.float32),
        ],
    )(x, p1, degt, ws1, wn1, b1, wn2, ws2, b2)


def _tc2_body(hs_r, p2_r, degt_r, out_r):
    deg = degt_r[:, 0] + degt_r[:, 1]
    rdeg = 1.0 / jnp.maximum(deg, 1.0)
    out_r[...] = hs_r[...] + (p2_r[0] + p2_r[1]) * rdeg[:, None]


def _tc2(hs, p2, degt):
    return pl.pallas_call(
        _tc2_body,
        grid=(GRID,),
        in_specs=[
            pl.BlockSpec((R, F), lambda i: (i, 0)),
            pl.BlockSpec((2, R, F), lambda i: (0, i, 0)),
            pl.BlockSpec((R, 2), lambda i: (i, 0)),
        ],
        out_specs=pl.BlockSpec((R, F), lambda i: (i, 0)),
        out_shape=jax.ShapeDtypeStruct((N, F), jnp.float32),
    )(hs, p2, degt)


def kernel(features, edge_index, W_self1, W_neigh1, b1, W_self2, W_neigh2, b2):
    src = edge_index[0].astype(jnp.int32)
    dst = edge_index[1].astype(jnp.int32)
    zrows = jnp.zeros((RPT, F), jnp.float32)

    p1, pdeg = _seg1(features, src, dst, zrows)
    degt = jnp.transpose(pdeg.reshape(2, NPAD)[:, :N])
    hs, y = _tc1(features, p1, degt, W_self1,
                 W_neigh1, b1.reshape(1, H), W_neigh2, W_self2,
                 b2.reshape(1, F))

    p2 = _seg2(y, src, dst, zrows)
    out = _tc2(hs, p2, degt)
    return out
